# baseline (speedup 1.0000x reference)
import functools

import jax
import jax.numpy as jnp
from jax import lax
from jax.experimental import pallas as pl
from jax.experimental.pallas import tpu as pltpu
from jax.experimental.pallas import tpu_sc as plsc

DEG = 32
F = 128
N_WORKERS = 32

NS_SC = 2560
NODES_PW = NS_SC // N_WORKERS
CHUNK = 8

TC1_BLOCK = 744
TC2_BLOCK = 512


def _agg1_body(h_ref, nei_ref, wt_ref, b_ref, out_ref, *, inv_count):
    s = jnp.sum(nei_ref[...], axis=1) + h_ref[...]
    agg = s * inv_count
    out_ref[...] = (
        jnp.dot(agg, wt_ref[...], preferred_element_type=jnp.float32) + b_ref[...]
    )


def _agg2_body(h_ref, part_ref, wt_ref, b_ref, out_ref, *, inv_count):
    agg = (part_ref[...] + h_ref[...]) * inv_count
    out_ref[...] = (
        jnp.dot(agg, wt_ref[...], preferred_element_type=jnp.float32) + b_ref[...]
    )


def _sc_mailbox_sum(nei2, nt):
    mesh = plsc.VectorSubcoreMesh(core_axis_name="c", subcore_axis_name="s")

    @functools.partial(
        pl.kernel,
        out_type=jax.ShapeDtypeStruct((NS_SC, F), jnp.float32),
        mesh=mesh,
        scratch_types=[
            pltpu.VMEM((CHUNK * DEG, F), jnp.float32),
            pltpu.VMEM((CHUNK, F), jnp.float32),
            pltpu.SemaphoreType.DMA,
        ],
    )
    def sc_sum(nei_hbm, out_hbm, buf, obuf, sem):
        wid = lax.axis_index("s") * 2 + lax.axis_index("c")
        base = wid * NODES_PW

        @pl.loop(0, NODES_PW // CHUNK)
        def _(step):
            node0 = base + step * CHUNK
            pltpu.async_copy(
                nei_hbm.at[pl.ds((nt + node0) * DEG, CHUNK * DEG)], buf, sem
            ).wait()

            @pl.loop(0, CHUNK)
            def _(g):
                for c1 in range(0, F, 16):
                    acc = buf.at[pl.ds(g * DEG, 1), pl.ds(c1, 16)][...]
                    for d in range(1, DEG):
                        acc = acc + buf.at[pl.ds(g * DEG + d, 1), pl.ds(c1, 16)][...]
                    obuf.at[pl.ds(g, 1), pl.ds(c1, 16)][...] = acc

            pltpu.async_copy(obuf, out_hbm.at[pl.ds(node0, CHUNK)], sem).wait()

    return sc_sum(nei2)


@jax.jit
def kernel(h, nei, W, b):
    n, in_feats = h.shape
    deg = nei.shape[1]
    out_feats = W.shape[0]
    nt = n - NS_SC
    inv_count = float(1.0 / (deg + 1))

    wt = W.T
    b2 = b.reshape(1, out_feats)
    nei2 = nei.reshape(n * deg, in_feats)

    part = _sc_mailbox_sum(nei2, nt)

    body1 = functools.partial(_agg1_body, inv_count=inv_count)
    out1 = pl.pallas_call(
        body1,
        grid=(nt // TC1_BLOCK,),
        in_specs=[
            pl.BlockSpec((TC1_BLOCK, in_feats), lambda i: (i, 0)),
            pl.BlockSpec((TC1_BLOCK, deg, in_feats), lambda i: (i, 0, 0)),
            pl.BlockSpec((in_feats, out_feats), lambda i: (0, 0)),
            pl.BlockSpec((1, out_feats), lambda i: (0, 0)),
        ],
        out_specs=pl.BlockSpec((TC1_BLOCK, out_feats), lambda i: (i, 0)),
        out_shape=jax.ShapeDtypeStruct((nt, out_feats), jnp.float32),
    )(h, nei, wt, b2)

    h_tail = lax.slice(h, (nt, 0), (n, in_feats))
    body2 = functools.partial(_agg2_body, inv_count=inv_count)
    out2 = pl.pallas_call(
        body2,
        grid=(NS_SC // TC2_BLOCK,),
        in_specs=[
            pl.BlockSpec((TC2_BLOCK, in_feats), lambda i: (i, 0)),
            pl.BlockSpec((TC2_BLOCK, in_feats), lambda i: (i, 0)),
            pl.BlockSpec((in_feats, out_feats), lambda i: (0, 0)),
            pl.BlockSpec((1, out_feats), lambda i: (0, 0)),
        ],
        out_specs=pl.BlockSpec((TC2_BLOCK, out_feats), lambda i: (i, 0)),
        out_shape=jax.ShapeDtypeStruct((NS_SC, out_feats), jnp.float32),
    )(h_tail, part, wt, b2)

    return jnp.concatenate([out1, out2], axis=0)


# speedup vs baseline: 1.0054x; 1.0054x over previous
import functools

import jax
import jax.numpy as jnp
from jax import lax
from jax.experimental import pallas as pl
from jax.experimental.pallas import tpu as pltpu
from jax.experimental.pallas import tpu_sc as plsc

DEG = 32
F = 128
N_WORKERS = 32

NS_SC = 2560
NODES_PW = NS_SC // N_WORKERS
CHUNK = 8

TC1_BLOCK = 744
TC2_BLOCK = 512


def _agg1_body(h_ref, nei_ref, wt_ref, b_ref, out_ref, *, inv_count):
    s = jnp.sum(nei_ref[...], axis=1) + h_ref[...]
    agg = s * inv_count
    out_ref[...] = (
        jnp.dot(agg, wt_ref[...], preferred_element_type=jnp.float32) + b_ref[...]
    )


def _agg2_body(h_ref, part_ref, wt_ref, b_ref, out_ref, *, inv_count):
    agg = (part_ref[...] + h_ref[...]) * inv_count
    out_ref[...] = (
        jnp.dot(agg, wt_ref[...], preferred_element_type=jnp.float32) + b_ref[...]
    )


def _sc_mailbox_sum(nei2, nt):
    mesh = plsc.VectorSubcoreMesh(core_axis_name="c", subcore_axis_name="s")

    nsteps = NODES_PW // CHUNK

    @functools.partial(
        pl.kernel,
        out_type=jax.ShapeDtypeStruct((NS_SC, F), jnp.float32),
        mesh=mesh,
        scratch_types=[
            pltpu.VMEM((CHUNK * DEG, F), jnp.float32),
            pltpu.VMEM((CHUNK * DEG, F), jnp.float32),
            pltpu.VMEM((CHUNK, F), jnp.float32),
            pltpu.VMEM((CHUNK, F), jnp.float32),
            pltpu.SemaphoreType.DMA,
            pltpu.SemaphoreType.DMA,
            pltpu.SemaphoreType.DMA,
            pltpu.SemaphoreType.DMA,
        ],
    )
    def sc_sum(nei_hbm, out_hbm, buf0, buf1, obuf0, obuf1, s0, s1, os0, os1):
        wid = lax.axis_index("s") * 2 + lax.axis_index("c")
        base = wid * NODES_PW

        def in_copy(step, buf, sem):
            node0 = base + step * CHUNK
            return pltpu.make_async_copy(
                nei_hbm.at[pl.ds((nt + node0) * DEG, CHUNK * DEG)], buf, sem
            )

        def out_copy(step, obuf, sem):
            node0 = base + step * CHUNK
            return pltpu.make_async_copy(obuf, out_hbm.at[pl.ds(node0, CHUNK)], sem)

        def reduce_chunk(buf, obuf):
            @pl.loop(0, CHUNK)
            def _(g):
                for c1 in range(0, F, 16):
                    acc = buf.at[pl.ds(g * DEG, 1), pl.ds(c1, 16)][...]
                    for d in range(1, DEG):
                        acc = acc + buf.at[pl.ds(g * DEG + d, 1), pl.ds(c1, 16)][...]
                    obuf.at[pl.ds(g, 1), pl.ds(c1, 16)][...] = acc

        in_copy(0, buf0, s0).start()

        @pl.loop(0, nsteps, step=2)
        def _(s):
            in_copy(s + 1, buf1, s1).start()
            in_copy(s, buf0, s0).wait()

            @pl.when(s >= 2)
            def _():
                out_copy(s - 2, obuf0, os0).wait()

            reduce_chunk(buf0, obuf0)
            out_copy(s, obuf0, os0).start()

            @pl.when(s + 2 < nsteps)
            def _():
                in_copy(s + 2, buf0, s0).start()

            in_copy(s + 1, buf1, s1).wait()

            @pl.when(s >= 2)
            def _():
                out_copy(s - 1, obuf1, os1).wait()

            reduce_chunk(buf1, obuf1)
            out_copy(s + 1, obuf1, os1).start()

        out_copy(nsteps - 2, obuf0, os0).wait()
        out_copy(nsteps - 1, obuf1, os1).wait()

    return sc_sum(nei2)


@jax.jit
def kernel(h, nei, W, b):
    n, in_feats = h.shape
    deg = nei.shape[1]
    out_feats = W.shape[0]
    nt = n - NS_SC
    inv_count = float(1.0 / (deg + 1))

    wt = W.T
    b2 = b.reshape(1, out_feats)
    nei2 = nei.reshape(n * deg, in_feats)

    part = _sc_mailbox_sum(nei2, nt)

    body1 = functools.partial(_agg1_body, inv_count=inv_count)
    out1 = pl.pallas_call(
        body1,
        grid=(nt // TC1_BLOCK,),
        in_specs=[
            pl.BlockSpec((TC1_BLOCK, in_feats), lambda i: (i, 0)),
            pl.BlockSpec((TC1_BLOCK, deg, in_feats), lambda i: (i, 0, 0)),
            pl.BlockSpec((in_feats, out_feats), lambda i: (0, 0)),
            pl.BlockSpec((1, out_feats), lambda i: (0, 0)),
        ],
        out_specs=pl.BlockSpec((TC1_BLOCK, out_feats), lambda i: (i, 0)),
        out_shape=jax.ShapeDtypeStruct((nt, out_feats), jnp.float32),
    )(h, nei, wt, b2)

    h_tail = lax.slice(h, (nt, 0), (n, in_feats))
    body2 = functools.partial(_agg2_body, inv_count=inv_count)
    out2 = pl.pallas_call(
        body2,
        grid=(NS_SC // TC2_BLOCK,),
        in_specs=[
            pl.BlockSpec((TC2_BLOCK, in_feats), lambda i: (i, 0)),
            pl.BlockSpec((TC2_BLOCK, in_feats), lambda i: (i, 0)),
            pl.BlockSpec((in_feats, out_feats), lambda i: (0, 0)),
            pl.BlockSpec((1, out_feats), lambda i: (0, 0)),
        ],
        out_specs=pl.BlockSpec((TC2_BLOCK, out_feats), lambda i: (i, 0)),
        out_shape=jax.ShapeDtypeStruct((NS_SC, out_feats), jnp.float32),
    )(h_tail, part, wt, b2)

    return jnp.concatenate([out1, out2], axis=0)


# speedup vs baseline: 1.0063x; 1.0009x over previous
import functools

import jax
import jax.numpy as jnp
from jax import lax
from jax.experimental import pallas as pl
from jax.experimental.pallas import tpu as pltpu
from jax.experimental.pallas import tpu_sc as plsc

DEG = 32
F = 128
N_WORKERS = 32

NS_SC = 2560
NODES_PW = NS_SC // N_WORKERS
CHUNK = 8

TC1_BLOCK = 744
TC2_BLOCK = 512


def _agg1_body(h_ref, nei_ref, wt_ref, b_ref, out_ref, *, inv_count):
    s = jnp.sum(nei_ref[...], axis=1) + h_ref[...]
    agg = s * inv_count
    out_ref[...] = (
        jnp.dot(agg, wt_ref[...], preferred_element_type=jnp.float32) + b_ref[...]
    )


def _agg2_body(h_ref, part_ref, wt_ref, b_ref, out_ref, *, inv_count):
    agg = (part_ref[...] + h_ref[...]) * inv_count
    out_ref[...] = (
        jnp.dot(agg, wt_ref[...], preferred_element_type=jnp.float32) + b_ref[...]
    )


def _sc_mailbox_sum(nei2, nt):
    mesh = plsc.VectorSubcoreMesh(core_axis_name="c", subcore_axis_name="s")

    nsteps = NODES_PW // CHUNK

    @functools.partial(
        pl.kernel,
        out_type=jax.ShapeDtypeStruct((NS_SC, F), jnp.float32),
        mesh=mesh,
        scratch_types=[
            pltpu.VMEM((CHUNK * DEG, F), jnp.float32),
            pltpu.VMEM((CHUNK * DEG, F), jnp.float32),
            pltpu.VMEM((CHUNK, F), jnp.float32),
            pltpu.VMEM((CHUNK, F), jnp.float32),
            pltpu.SemaphoreType.DMA,
            pltpu.SemaphoreType.DMA,
            pltpu.SemaphoreType.DMA,
            pltpu.SemaphoreType.DMA,
        ],
    )
    def sc_sum(nei_hbm, out_hbm, buf0, buf1, obuf0, obuf1, s0, s1, os0, os1):
        wid = lax.axis_index("s") * 2 + lax.axis_index("c")
        base = wid * NODES_PW

        def in_copy(step, buf, sem):
            node0 = base + step * CHUNK
            return pltpu.make_async_copy(
                nei_hbm.at[pl.ds((nt + node0) * DEG, CHUNK * DEG)], buf, sem
            )

        def out_copy(step, obuf, sem):
            node0 = base + step * CHUNK
            return pltpu.make_async_copy(obuf, out_hbm.at[pl.ds(node0, CHUNK)], sem)

        def reduce_chunk(buf, obuf):
            @pl.loop(0, CHUNK)
            def _(g):
                for c1 in range(0, F, 16):
                    vals = [
                        buf.at[pl.ds(g * DEG + d, 1), pl.ds(c1, 16)][...]
                        for d in range(DEG)
                    ]
                    while len(vals) > 1:
                        nxt = [
                            vals[i] + vals[i + 1] for i in range(0, len(vals) - 1, 2)
                        ]
                        if len(vals) % 2:
                            nxt.append(vals[-1])
                        vals = nxt
                    obuf.at[pl.ds(g, 1), pl.ds(c1, 16)][...] = vals[0]

        in_copy(0, buf0, s0).start()

        @pl.loop(0, nsteps, step=2)
        def _(s):
            in_copy(s + 1, buf1, s1).start()
            in_copy(s, buf0, s0).wait()

            @pl.when(s >= 2)
            def _():
                out_copy(s - 2, obuf0, os0).wait()

            reduce_chunk(buf0, obuf0)
            out_copy(s, obuf0, os0).start()

            @pl.when(s + 2 < nsteps)
            def _():
                in_copy(s + 2, buf0, s0).start()

            in_copy(s + 1, buf1, s1).wait()

            @pl.when(s >= 2)
            def _():
                out_copy(s - 1, obuf1, os1).wait()

            reduce_chunk(buf1, obuf1)
            out_copy(s + 1, obuf1, os1).start()

        out_copy(nsteps - 2, obuf0, os0).wait()
        out_copy(nsteps - 1, obuf1, os1).wait()

    return sc_sum(nei2)


@jax.jit
def kernel(h, nei, W, b):
    n, in_feats = h.shape
    deg = nei.shape[1]
    out_feats = W.shape[0]
    nt = n - NS_SC
    inv_count = float(1.0 / (deg + 1))

    wt = W.T
    b2 = b.reshape(1, out_feats)
    nei2 = nei.reshape(n * deg, in_feats)

    part = _sc_mailbox_sum(nei2, nt)

    body1 = functools.partial(_agg1_body, inv_count=inv_count)
    out1 = pl.pallas_call(
        body1,
        grid=(nt // TC1_BLOCK,),
        in_specs=[
            pl.BlockSpec((TC1_BLOCK, in_feats), lambda i: (i, 0)),
            pl.BlockSpec((TC1_BLOCK, deg, in_feats), lambda i: (i, 0, 0)),
            pl.BlockSpec((in_feats, out_feats), lambda i: (0, 0)),
            pl.BlockSpec((1, out_feats), lambda i: (0, 0)),
        ],
        out_specs=pl.BlockSpec((TC1_BLOCK, out_feats), lambda i: (i, 0)),
        out_shape=jax.ShapeDtypeStruct((nt, out_feats), jnp.float32),
    )(h, nei, wt, b2)

    h_tail = lax.slice(h, (nt, 0), (n, in_feats))
    body2 = functools.partial(_agg2_body, inv_count=inv_count)
    out2 = pl.pallas_call(
        body2,
        grid=(NS_SC // TC2_BLOCK,),
        in_specs=[
            pl.BlockSpec((TC2_BLOCK, in_feats), lambda i: (i, 0)),
            pl.BlockSpec((TC2_BLOCK, in_feats), lambda i: (i, 0)),
            pl.BlockSpec((in_feats, out_feats), lambda i: (0, 0)),
            pl.BlockSpec((1, out_feats), lambda i: (0, 0)),
        ],
        out_specs=pl.BlockSpec((TC2_BLOCK, out_feats), lambda i: (i, 0)),
        out_shape=jax.ShapeDtypeStruct((NS_SC, out_feats), jnp.float32),
    )(h_tail, part, wt, b2)

    return jnp.concatenate([out1, out2], axis=0)


# speedup vs baseline: 1.2676x; 1.2597x over previous
import functools

import jax
import jax.numpy as jnp
from jax.experimental import pallas as pl


def _agg_body(h_ref, nei_ref, wt_ref, b_ref, out_ref, *, inv_count):
    s = jnp.sum(nei_ref[0], axis=1) + h_ref[0]
    agg = s * inv_count
    out_ref[0] = (
        jnp.dot(agg, wt_ref[...], preferred_element_type=jnp.float32) + b_ref[...]
    )


@jax.jit
def kernel(h, nei, W, b):
    n, in_feats = h.shape
    deg = nei.shape[1]
    out_feats = W.shape[0]

    block = 500
    grid = (n // block,)
    g = n // block

    h3 = h.reshape(g, block, in_feats)
    nei4 = nei.reshape(g, block, deg, in_feats)
    wt = W.T
    b2 = b.reshape(1, out_feats)

    body = functools.partial(_agg_body, inv_count=float(1.0 / (deg + 1)))

    out = pl.pallas_call(
        body,
        grid=grid,
        in_specs=[
            pl.BlockSpec((1, block, in_feats), lambda i: (i, 0, 0)),
            pl.BlockSpec((1, block, deg, in_feats), lambda i: (i, 0, 0, 0)),
            pl.BlockSpec((in_feats, out_feats), lambda i: (0, 0)),
            pl.BlockSpec((1, out_feats), lambda i: (0, 0)),
        ],
        out_specs=pl.BlockSpec((1, block, out_feats), lambda i: (i, 0, 0)),
        out_shape=jax.ShapeDtypeStruct((g, block, out_feats), jnp.float32),
    )(h3, nei4, wt, b2)
    return out.reshape(n, out_feats)


# speedup vs baseline: 1.5173x; 1.1970x over previous
import functools

import jax
import jax.numpy as jnp
from jax.experimental import pallas as pl


def _agg_body(h_ref, nei_ref, wt_ref, b_ref, out_ref, *, inv_count):
    s = jnp.sum(nei_ref[...], axis=1) + h_ref[...]
    agg = s * inv_count
    out_ref[...] = (
        jnp.dot(agg, wt_ref[...], preferred_element_type=jnp.float32) + b_ref[...]
    )


@jax.jit
def kernel(h, nei, W, b):
    n, in_feats = h.shape
    deg = nei.shape[1]
    out_feats = W.shape[0]

    block = 512
    grid = (pl.cdiv(n, block),)

    wt = W.T
    b2 = b.reshape(1, out_feats)

    body = functools.partial(_agg_body, inv_count=float(1.0 / (deg + 1)))

    return pl.pallas_call(
        body,
        grid=grid,
        in_specs=[
            pl.BlockSpec((block, in_feats), lambda i: (i, 0)),
            pl.BlockSpec((block, deg, in_feats), lambda i: (i, 0, 0)),
            pl.BlockSpec((in_feats, out_feats), lambda i: (0, 0)),
            pl.BlockSpec((1, out_feats), lambda i: (0, 0)),
        ],
        out_specs=pl.BlockSpec((block, out_feats), lambda i: (i, 0)),
        out_shape=jax.ShapeDtypeStruct((n, out_feats), jnp.float32),
    )(h, nei, wt, b2)


# speedup vs baseline: 1.5182x; 1.0006x over previous
import functools

import jax
import jax.numpy as jnp
from jax.experimental import pallas as pl


def _agg_body(h_ref, nei_ref, wt_ref, b_ref, out_ref, *, inv_count):
    s = jnp.sum(nei_ref[...], axis=1) + h_ref[...]
    agg = s * inv_count
    out_ref[...] = (
        jnp.dot(agg, wt_ref[...], preferred_element_type=jnp.float32) + b_ref[...]
    )


@jax.jit
def kernel(h, nei, W, b):
    n, in_feats = h.shape
    deg = nei.shape[1]
    out_feats = W.shape[0]

    block = 448
    grid = (pl.cdiv(n, block),)

    wt = W.T
    b2 = b.reshape(1, out_feats)

    body = functools.partial(_agg_body, inv_count=float(1.0 / (deg + 1)))

    return pl.pallas_call(
        body,
        grid=grid,
        in_specs=[
            pl.BlockSpec((block, in_feats), lambda i: (i, 0)),
            pl.BlockSpec((block, deg, in_feats), lambda i: (i, 0, 0)),
            pl.BlockSpec((in_feats, out_feats), lambda i: (0, 0)),
            pl.BlockSpec((1, out_feats), lambda i: (0, 0)),
        ],
        out_specs=pl.BlockSpec((block, out_feats), lambda i: (i, 0)),
        out_shape=jax.ShapeDtypeStruct((n, out_feats), jnp.float32),
    )(h, nei, wt, b2)
